# Initial kernel scaffold; baseline (speedup 1.0000x reference)
import jax
import jax.numpy as jnp
from jax import lax
from jax.experimental import pallas as pl
from jax.experimental.pallas import tpu as pltpu
from jax.experimental.pallas import tpu_sc as plsc

VOCAB = 1000000
EMBED_DIM = 32
BATCH = 16384
HIST = 200

_NC = 2
_NS = 16
_NW = _NC * _NS

_B = BATCH * HIST
_PER_W = _B // _NW
_BLK = 2048
_NBLK = _PER_W // _BLK


def _gather_body(x_hbm, table_hbm, out_hbm, idx_v, rows_v, sem):
    wid = lax.axis_index("s") * _NC + lax.axis_index("c")
    base = wid * _PER_W

    def blk(i, carry):
        off = base + i * _BLK
        pltpu.sync_copy(x_hbm.at[pl.ds(off, _BLK)], idx_v)
        pltpu.async_copy(table_hbm.at[idx_v], rows_v, sem).wait()
        pltpu.sync_copy(rows_v, out_hbm.at[pl.ds(off, _BLK)])
        return carry

    lax.fori_loop(0, _NBLK, blk, 0)


@jax.jit
def _run(x_flat, table):
    mesh = plsc.VectorSubcoreMesh(core_axis_name="c", subcore_axis_name="s")
    f = pl.kernel(
        _gather_body,
        out_type=jax.ShapeDtypeStruct((_B, EMBED_DIM), jnp.float32),
        mesh=mesh,
        scratch_types=[
            pltpu.VMEM((_BLK,), jnp.int32),
            pltpu.VMEM((_BLK, EMBED_DIM), jnp.float32),
            pltpu.SemaphoreType.DMA,
        ],
    )
    return f(x_flat, table)


def kernel(x, table):
    out = _run(x.reshape(_B), table)
    return out.reshape(BATCH, HIST, EMBED_DIM)


# speedup vs baseline: 4.9506x; 4.9506x over previous
import jax
import jax.numpy as jnp
from jax import lax
from jax.experimental import pallas as pl
from jax.experimental.pallas import tpu as pltpu
from jax.experimental.pallas import tpu_sc as plsc

VOCAB = 1000000
EMBED_DIM = 32
BATCH = 16384
HIST = 200

_NC = 2
_NS = 16
_NW = _NC * _NS

_B = BATCH * HIST
_PER_W = _B // _NW
_BLK = 2048
_NBLK = _PER_W // _BLK


def _gather_body(x_hbm, table_hbm, out_hbm, idx_v, rows_v, sem):
    wid = lax.axis_index("s") * _NC + lax.axis_index("c")
    base = wid * _PER_W

    def blk(i, carry):
        off = base + i * _BLK
        pltpu.sync_copy(x_hbm.at[pl.ds(off, _BLK)], idx_v)
        pltpu.async_copy(table_hbm.at[idx_v], rows_v, sem).wait()
        pltpu.sync_copy(rows_v, out_hbm.at[pl.ds(off, _BLK)])
        return carry

    lax.fori_loop(0, _NBLK, blk, 0)


@jax.jit
def _run(x_flat, table):
    mesh = plsc.VectorSubcoreMesh(core_axis_name="c", subcore_axis_name="s")
    f = pl.kernel(
        _gather_body,
        out_type=jax.ShapeDtypeStruct((_B, EMBED_DIM), jnp.float32),
        mesh=mesh,
        scratch_types=[
            pltpu.VMEM((_BLK,), jnp.int32),
            pltpu.VMEM((_BLK, EMBED_DIM), jnp.float32),
            pltpu.SemaphoreType.DMA,
        ],
        compiler_params=pltpu.CompilerParams(use_tc_tiling_on_sc=False),
    )
    return f(x_flat, table)


def kernel(x, table):
    out = _run(x.reshape(_B), table)
    return out.reshape(BATCH, HIST, EMBED_DIM)


# speedup vs baseline: 5.0500x; 1.0201x over previous
import jax
import jax.numpy as jnp
from jax import lax
from jax.experimental import pallas as pl
from jax.experimental.pallas import tpu as pltpu
from jax.experimental.pallas import tpu_sc as plsc

VOCAB = 1000000
EMBED_DIM = 32
BATCH = 16384
HIST = 200

_NC = 2
_NS = 16
_NW = _NC * _NS

_B = BATCH * HIST
_PER_W = _B // _NW
_BLK = 1600
_NBLK = _PER_W // _BLK

assert _PER_W % _BLK == 0 and _NBLK % 2 == 0 and _NBLK >= 4


def _gather_body(x_hbm, table_hbm, out_hbm,
                 idx0, idx1, rows0, rows1,
                 si0, si1, sg0, sg1, so0, so1):
    wid = lax.axis_index("s") * _NC + lax.axis_index("c")
    base = wid * _PER_W
    idx = (idx0, idx1)
    rows = (rows0, rows1)
    si = (si0, si1)
    sg = (sg0, sg1)
    so = (so0, so1)
    last = base + (_NBLK - 1) * _BLK

    def start_idx(j, b):
        off = jnp.minimum(base + j * _BLK, last)
        pltpu.async_copy(x_hbm.at[pl.ds(off, _BLK)], idx[b], si[b])

    def wait_idx(b):
        pltpu.make_async_copy(x_hbm.at[pl.ds(base, _BLK)], idx[b], si[b]).wait()

    def start_gather(b):
        pltpu.async_copy(table_hbm.at[idx[b]], rows[b], sg[b])

    def wait_gather(b):
        pltpu.make_async_copy(table_hbm.at[idx[b]], rows[b], sg[b]).wait()

    def start_store(j, b):
        off = base + j * _BLK
        pltpu.async_copy(rows[b], out_hbm.at[pl.ds(off, _BLK)], so[b])

    def wait_store(b):
        pltpu.make_async_copy(rows[b], out_hbm.at[pl.ds(base, _BLK)], so[b]).wait()

    start_idx(0, 0)
    wait_idx(0)
    start_gather(0)
    start_idx(1, 1)
    wait_idx(1)
    start_gather(1)
    wait_gather(0)
    start_store(0, 0)
    start_idx(2, 0)

    def group(g, carry):
        for b in (0, 1):
            j = 2 * g + b
            wait_idx(b)
            wait_store(b)
            start_gather(b)
            wait_gather(1 - b)
            start_store(j - 1, 1 - b)
            start_idx(j + 1, 1 - b)
        return carry

    lax.fori_loop(1, _NBLK // 2, group, 0)

    wait_gather(1)
    start_store(_NBLK - 1, 1)
    wait_idx(0)
    wait_store(0)
    wait_store(1)


@jax.jit
def _run(x_flat, table):
    mesh = plsc.VectorSubcoreMesh(core_axis_name="c", subcore_axis_name="s")
    f = pl.kernel(
        _gather_body,
        out_type=jax.ShapeDtypeStruct((_B, EMBED_DIM), jnp.float32),
        mesh=mesh,
        scratch_types=[
            pltpu.VMEM((_BLK,), jnp.int32),
            pltpu.VMEM((_BLK,), jnp.int32),
            pltpu.VMEM((_BLK, EMBED_DIM), jnp.float32),
            pltpu.VMEM((_BLK, EMBED_DIM), jnp.float32),
            pltpu.SemaphoreType.DMA,
            pltpu.SemaphoreType.DMA,
            pltpu.SemaphoreType.DMA,
            pltpu.SemaphoreType.DMA,
            pltpu.SemaphoreType.DMA,
            pltpu.SemaphoreType.DMA,
        ],
        compiler_params=pltpu.CompilerParams(use_tc_tiling_on_sc=False),
    )
    return f(x_flat, table)


def kernel(x, table):
    out = _run(x.reshape(_B), table)
    return out.reshape(BATCH, HIST, EMBED_DIM)


# speedup vs baseline: 5.0812x; 1.0062x over previous
import jax
import jax.numpy as jnp
from jax import lax
from jax.experimental import pallas as pl
from jax.experimental.pallas import tpu as pltpu
from jax.experimental.pallas import tpu_sc as plsc

VOCAB = 1000000
EMBED_DIM = 32
BATCH = 16384
HIST = 200

_NC = 2
_NS = 16
_NW = _NC * _NS
_HT = HIST // 8
_BTW = (BATCH // 128) // _NW
_NU = _HT * _BTW


def _gather_body(x2_hbm, table_hbm, out_hbm,
                 idx0, idx1, rows0, rows1, sg0, sg1, so0, so1):
    idxb = (idx0, idx1)
    rows = (rows0, rows1)
    sg = (sg0, sg1)
    so = (so0, so1)
    wid = lax.axis_index("s") * _NC + lax.axis_index("c")

    def unit_decode(t):
        btl = t // _HT
        ht = t % _HT
        return _BTW * wid + btl, ht

    def load_idx(t, p):
        bt, ht = unit_decode(t)
        pltpu.sync_copy(
            x2_hbm.at[pl.ds(8 * ht, 8), pl.ds(128 * bt, 128)], idxb[p])

    def fire_gathers(p):
        for h0 in range(8):
            pltpu.async_copy(table_hbm.at[idxb[p].at[h0]], rows[p].at[h0], sg[p])

    def wait_gathers(p):
        for h0 in range(8):
            pltpu.make_async_copy(
                table_hbm.at[idxb[p].at[h0]], rows[p].at[h0], sg[p]).wait()

    def fire_stores(t, p):
        bt, ht = unit_decode(t)
        for h0 in range(8):
            pltpu.async_copy(
                rows[p].at[h0],
                out_hbm.at[pl.ds(bt * 128, 128), 8 * ht + h0, :], so[p])

    def wait_stores(p):
        for h0 in range(8):
            pltpu.make_async_copy(
                rows[p].at[h0],
                out_hbm.at[pl.ds(0, 128), h0, :], so[p]).wait()

    load_idx(0, 0)
    fire_gathers(0)
    load_idx(1, 1)
    fire_gathers(1)
    wait_gathers(0)
    fire_stores(0, 0)

    def group(g, c):
        for p in (0, 1):
            t = 2 * g + p
            load_idx(t, p)
            wait_stores(p)
            fire_gathers(p)
            wait_gathers(1 - p)
            fire_stores(t - 1, 1 - p)
        return c

    lax.fori_loop(1, _NU // 2, group, 0)

    wait_gathers(1)
    fire_stores(_NU - 1, 1)
    wait_stores(0)
    wait_stores(1)


@jax.jit
def _run(x_t, table):
    mesh = plsc.VectorSubcoreMesh(core_axis_name="c", subcore_axis_name="s")
    f = pl.kernel(
        _gather_body,
        out_type=jax.ShapeDtypeStruct((BATCH, HIST, EMBED_DIM), jnp.float32),
        mesh=mesh,
        scratch_types=[
            pltpu.VMEM((8, 128), jnp.int32),
            pltpu.VMEM((8, 128), jnp.int32),
            pltpu.VMEM((8, 128, EMBED_DIM), jnp.float32),
            pltpu.VMEM((8, 128, EMBED_DIM), jnp.float32),
            pltpu.SemaphoreType.DMA,
            pltpu.SemaphoreType.DMA,
            pltpu.SemaphoreType.DMA,
            pltpu.SemaphoreType.DMA,
        ],
        compiler_params=pltpu.CompilerParams(use_tc_tiling_on_sc=False),
    )
    return f(x_t, table)


def kernel(x, table):
    return _run(x.T, table)


# speedup vs baseline: 5.0982x; 1.0033x over previous
import jax
import jax.numpy as jnp
from jax import lax
from jax.experimental import pallas as pl
from jax.experimental.pallas import tpu as pltpu
from jax.experimental.pallas import tpu_sc as plsc

VOCAB = 1000000
EMBED_DIM = 32
BATCH = 16384
HIST = 200

_NC = 2
_NS = 16
_NW = _NC * _NS
_HT = HIST // 8
_BTW = (BATCH // 128) // _NW
_NU = _HT * _BTW


def _gather_body(x2_hbm, table_hbm, out_hbm,
                 idx0, idx1, rows0, rows1, sg0, sg1, so0, so1):
    idxb = (idx0, idx1)
    rows = (rows0, rows1)
    sg = (sg0, sg1)
    so = (so0, so1)
    wid = lax.axis_index("s") * _NC + lax.axis_index("c")

    def unit_decode(t):
        btl = t // _HT
        ht = t % _HT
        return _BTW * wid + btl, ht

    def load_idx(t, p):
        bt, ht = unit_decode(t)
        pltpu.sync_copy(x2_hbm.at[pl.ds((ht * 128 + bt) * 8, 8), :], idxb[p])

    def fire_gathers(p):
        for h0 in range(8):
            pltpu.async_copy(table_hbm.at[idxb[p].at[h0]], rows[p].at[h0], sg[p])

    def wait_gathers(p):
        for h0 in range(8):
            pltpu.make_async_copy(
                table_hbm.at[idxb[p].at[h0]], rows[p].at[h0], sg[p]).wait()

    def fire_stores(t, p):
        bt, ht = unit_decode(t)
        for h0 in range(8):
            pltpu.async_copy(
                rows[p].at[h0],
                out_hbm.at[pl.ds(bt * 128, 128), 8 * ht + h0, :], so[p])

    def wait_stores(p):
        for h0 in range(8):
            pltpu.make_async_copy(
                rows[p].at[h0],
                out_hbm.at[pl.ds(0, 128), h0, :], so[p]).wait()

    load_idx(0, 0)
    fire_gathers(0)
    load_idx(1, 1)
    fire_gathers(1)
    wait_gathers(0)
    fire_stores(0, 0)

    def group(g, c):
        for p in (0, 1):
            t = 2 * g + p
            load_idx(t, p)
            wait_stores(p)
            fire_gathers(p)
            wait_gathers(1 - p)
            fire_stores(t - 1, 1 - p)
        return c

    lax.fori_loop(1, _NU // 2, group, 0)

    wait_gathers(1)
    fire_stores(_NU - 1, 1)
    wait_stores(0)
    wait_stores(1)


@jax.jit
def _run(x2d, table):
    mesh = plsc.VectorSubcoreMesh(core_axis_name="c", subcore_axis_name="s")
    f = pl.kernel(
        _gather_body,
        out_type=jax.ShapeDtypeStruct((BATCH, HIST, EMBED_DIM), jnp.float32),
        mesh=mesh,
        scratch_types=[
            pltpu.VMEM((8, 128), jnp.int32),
            pltpu.VMEM((8, 128), jnp.int32),
            pltpu.VMEM((8, 128, EMBED_DIM), jnp.float32),
            pltpu.VMEM((8, 128, EMBED_DIM), jnp.float32),
            pltpu.SemaphoreType.DMA,
            pltpu.SemaphoreType.DMA,
            pltpu.SemaphoreType.DMA,
            pltpu.SemaphoreType.DMA,
        ],
        compiler_params=pltpu.CompilerParams(use_tc_tiling_on_sc=False),
    )
    return f(x2d, table)


def kernel(x, table):
    x2d = (x.T.reshape(_HT, 8, 128, 128)
           .transpose((0, 2, 1, 3))
           .reshape(_HT * 1024, 128))
    return _run(x2d, table)
